# baseline (speedup 1.0000x reference)
import math

import jax
import jax.numpy as jnp
from jax.experimental import pallas as pl
from jax.experimental.pallas import tpu as pltpu

EMBED_DIM = 512
MAX_SEQ_LEN = 1024
HALF_DIM = EMBED_DIM // 2
SCALE = math.log(10000.0) / HALF_DIM
BLOCK_ROWS = 128
GRID = MAX_SEQ_LEN // BLOCK_ROWS
HALF_PI = math.pi / 2.0


def _inv_freq(rows):
    col = jax.lax.broadcasted_iota(jnp.int32, (rows, EMBED_DIM), 1)
    k = jnp.right_shift(col, 1).astype(jnp.float32)
    return col, jnp.exp(k * (-SCALE))


def _pos_table_body(out_ref, cb_ref, sb_ref):
    g = pl.program_id(0)

    @pl.when(g == 0)
    def _build_fine_tables():
        _, invf8 = _inv_freq(8)
        c_row = jax.lax.broadcasted_iota(
            jnp.int32, (8, EMBED_DIM), 0).astype(jnp.float32)
        ang_c = c_row * invf8
        s_c, c_c = jnp.sin(ang_c), jnp.cos(ang_c)
        _, invf4 = _inv_freq(4)
        b_row = jax.lax.broadcasted_iota(
            jnp.int32, (4, EMBED_DIM), 0).astype(jnp.float32)
        ang_b = (b_row * 8.0) * invf4
        s_b, c_b = jnp.sin(ang_b), jnp.cos(ang_b)
        cb3 = c_b[:, None, :] * c_c[None, :, :] - s_b[:, None, :] * s_c[None, :, :]
        sb3 = s_b[:, None, :] * c_c[None, :, :] + c_b[:, None, :] * s_c[None, :, :]
        cb_ref[...] = cb3.reshape(32, EMBED_DIM)
        sb_ref[...] = sb3.reshape(32, EMBED_DIM)

    col, invf4 = _inv_freq(4)
    phase = jnp.where((col & 1) == 1, HALF_PI, 0.0)
    a_row = jax.lax.broadcasted_iota(
        jnp.int32, (4, EMBED_DIM), 0).astype(jnp.float32)
    base = (g * BLOCK_ROWS).astype(jnp.float32)
    ang_a = (base + a_row * 32.0) * invf4 + phase
    x = jnp.sin(ang_a)
    y = jnp.cos(ang_a)
    cb = cb_ref[...]
    sb = sb_ref[...]
    out3 = x[:, None, :] * cb[None, :, :] + y[:, None, :] * sb[None, :, :]
    out_ref[...] = out3.reshape(BLOCK_ROWS, EMBED_DIM)


def kernel(seq_embeds):
    del seq_embeds
    return pl.pallas_call(
        _pos_table_body,
        grid=(GRID,),
        out_specs=pl.BlockSpec((BLOCK_ROWS, EMBED_DIM), lambda g: (g, 0)),
        out_shape=jax.ShapeDtypeStruct((MAX_SEQ_LEN, EMBED_DIM), jnp.float32),
        scratch_shapes=[
            pltpu.VMEM((32, EMBED_DIM), jnp.float32),
            pltpu.VMEM((32, EMBED_DIM), jnp.float32),
        ],
    )()


# speedup vs baseline: 2.1901x; 2.1901x over previous
import math

import jax
import jax.numpy as jnp
from jax.experimental import pallas as pl
from jax.experimental.pallas import tpu as pltpu

EMBED_DIM = 512
MAX_SEQ_LEN = 1024
HALF_DIM = EMBED_DIM // 2
SCALE = math.log(10000.0) / HALF_DIM
HALF_PI = math.pi / 2.0
NCHUNK = 4
CHUNK_ROWS = MAX_SEQ_LEN // NCHUNK


def _inv_freq(rows):
    col = jax.lax.broadcasted_iota(jnp.int32, (rows, EMBED_DIM), 1)
    k = jnp.right_shift(col, 1).astype(jnp.float32)
    return col, jnp.exp(k * (-SCALE))


def _pos_table_body(out_hbm, buf, sems):
    _, invf8 = _inv_freq(8)
    c_row = jax.lax.broadcasted_iota(
        jnp.int32, (8, EMBED_DIM), 0).astype(jnp.float32)
    ang_c = c_row * invf8
    s_c, c_c = jnp.sin(ang_c), jnp.cos(ang_c)
    col4, invf4 = _inv_freq(4)
    b_row = jax.lax.broadcasted_iota(
        jnp.int32, (4, EMBED_DIM), 0).astype(jnp.float32)
    ang_b = (b_row * 8.0) * invf4
    s_b, c_b = jnp.sin(ang_b), jnp.cos(ang_b)
    cb = (c_b[:, None, :] * c_c[None, :, :]
          - s_b[:, None, :] * s_c[None, :, :]).reshape(32, EMBED_DIM)
    sb = (s_b[:, None, :] * c_c[None, :, :]
          + c_b[:, None, :] * s_c[None, :, :]).reshape(32, EMBED_DIM)

    col8, invf8b = _inv_freq(8)
    phase = jnp.where((col8 & 1) == 1, HALF_PI, 0.0)
    a_row = jax.lax.broadcasted_iota(
        jnp.int32, (8, EMBED_DIM), 0).astype(jnp.float32)

    for chunk in range(NCHUNK):
        base = float(chunk * CHUNK_ROWS)
        ang_a = (base + a_row * 32.0) * invf8b + phase
        x = jnp.sin(ang_a)
        y = jnp.cos(ang_a)
        out3 = x[:, None, :] * cb[None, :, :] + y[:, None, :] * sb[None, :, :]
        lo = chunk * CHUNK_ROWS
        buf[pl.ds(lo, CHUNK_ROWS), :] = out3.reshape(CHUNK_ROWS, EMBED_DIM)
        pltpu.make_async_copy(
            buf.at[pl.ds(lo, CHUNK_ROWS), :],
            out_hbm.at[pl.ds(lo, CHUNK_ROWS), :],
            sems.at[chunk],
        ).start()

    for chunk in range(NCHUNK):
        lo = chunk * CHUNK_ROWS
        pltpu.make_async_copy(
            buf.at[pl.ds(lo, CHUNK_ROWS), :],
            out_hbm.at[pl.ds(lo, CHUNK_ROWS), :],
            sems.at[chunk],
        ).wait()


def kernel(seq_embeds):
    del seq_embeds
    return pl.pallas_call(
        _pos_table_body,
        out_specs=pl.BlockSpec(memory_space=pl.ANY),
        out_shape=jax.ShapeDtypeStruct((MAX_SEQ_LEN, EMBED_DIM), jnp.float32),
        scratch_shapes=[
            pltpu.VMEM((MAX_SEQ_LEN, EMBED_DIM), jnp.float32),
            pltpu.SemaphoreType.DMA((NCHUNK,)),
        ],
    )()
